# Initial kernel scaffold; baseline (speedup 1.0000x reference)
import jax
import jax.numpy as jnp
from jax.experimental import pallas as pl


def kernel(x, adj, params):
    raise NotImplementedError("write your pallas kernel here")


# speedup vs baseline: 1.1141x; 1.1141x over previous
import jax
import jax.numpy as jnp
from jax.experimental import pallas as pl
from jax.experimental.pallas import tpu as pltpu

_N = 2048
_D = 64


def _gcn_body(x_ref, adj_ref, w1_ref, b1_ref, w2_ref, b2_ref,
              v1_ref, c1_ref, v2_ref, c2_ref, fcw_ref, fcb_ref, out_ref):
    adj = adj_ref[0]
    h = x_ref[0]

    def gconv(h, w1, b1, w2, b2):
        agg = jnp.dot(adj, h, preferred_element_type=jnp.float32)
        return (jnp.dot(h, w1, preferred_element_type=jnp.float32) + b1
                + jnp.dot(agg, w2, preferred_element_type=jnp.float32) + b2)

    h = gconv(h, w1_ref[0], b1_ref[0], w2_ref[0], b2_ref[0])
    for r in range(6):
        i, j = 1 + 2 * r, 2 + 2 * r
        o1 = jnp.maximum(
            gconv(h, w1_ref[i], b1_ref[i], w2_ref[i], b2_ref[i]), 0.0)
        h = jnp.maximum(
            gconv(o1, w1_ref[j], b1_ref[j], w2_ref[j], b2_ref[j]) + h, 0.0)
    g = gconv(h, v1_ref[0], c1_ref[0], v2_ref[0], c2_ref[0])
    out_ref[0] = (jnp.dot(g, fcw_ref[...], preferred_element_type=jnp.float32)
                  + fcb_ref[...])


def kernel(x, adj, params):
    B = x.shape[0]
    L = params["layers"]
    w1 = jnp.stack([l["W1"] for l in L[:13]])
    b1 = jnp.stack([l["b1"] for l in L[:13]])[:, None, :]
    w2 = jnp.stack([l["W2"] for l in L[:13]])
    b2 = jnp.stack([l["b2"] for l in L[:13]])[:, None, :]
    v1 = L[13]["W1"][None]
    c1 = L[13]["b1"][None, None, :]
    v2 = L[13]["W2"][None]
    c2 = L[13]["b2"][None, None, :]
    fcw = params["fcW"]
    fcb = params["fcb"][None, :]

    full = lambda s: pl.BlockSpec(s, lambda b: (0,) * len(s))
    grid_spec = pl.GridSpec(
        grid=(B,),
        in_specs=[
            pl.BlockSpec((1, _N, _D), lambda b: (b, 0, 0)),
            pl.BlockSpec((1, _N, _N), lambda b: (b, 0, 0)),
            full((13, _D, _D)), full((13, 1, _D)),
            full((13, _D, _D)), full((13, 1, _D)),
            full((1, _D, 32)), full((1, 1, 32)),
            full((1, _D, 32)), full((1, 1, 32)),
            full((32, 2)), full((1, 2)),
        ],
        out_specs=pl.BlockSpec((1, _N, 2), lambda b: (b, 0, 0)),
    )
    return pl.pallas_call(
        _gcn_body,
        grid_spec=grid_spec,
        out_shape=jax.ShapeDtypeStruct((B, _N, 2), jnp.float32),
        compiler_params=pltpu.CompilerParams(
            dimension_semantics=("arbitrary",),
            vmem_limit_bytes=100 * 1024 * 1024,
        ),
    )(x, adj, w1, b1, w2, b2, v1, c1, v2, c2, fcw, fcb)
